# baseline (speedup 1.0000x reference)
import functools

import jax
import jax.numpy as jnp
from jax import lax
from jax.experimental import pallas as pl
from jax.experimental.pallas import tpu as pltpu
from jax.experimental.pallas import tpu_sc as plsc

NUM_EMB = 1000000
EMB = 32
HID = 64
B = 4096
T = 50
TPAD = 56

NC = 2
NS = 16
NW = NC * NS

Q = 5
TQ = T // Q
NTOK = B * TQ
BPW = NTOK // NW
CHUNK = 256
NCHUNK = BPW // CHUNK
NBUF = 5


@functools.cache
def _make_gather(q):
    @functools.partial(
        pl.kernel,
        out_type=jax.ShapeDtypeStruct((NTOK, EMB), jnp.float32),
        mesh=plsc.VectorSubcoreMesh(core_axis_name="c", subcore_axis_name="s"),
        compiler_params=pltpu.CompilerParams(use_tc_tiling_on_sc=False),
        scratch_types=[
            pltpu.VMEM((2, B), jnp.int32),
            pltpu.VMEM((NBUF, CHUNK, EMB), jnp.float32),
            pltpu.SemaphoreType.DMA,
            pltpu.SemaphoreType.DMA,
        ],
    )
    def _gather(idx_hbm, table2d, out_hbm, idx_v, rows_v, gsem, osem):
        wid = lax.axis_index("s") * NC + lax.axis_index("c")
        gbase = q * NTOK + wid * BPW
        base = wid * BPW
        r0 = gbase // B
        pltpu.sync_copy(idx_hbm.at[pl.ds(r0, 2)], idx_v)

        def gather_chunk(i):
            p = gbase + i * CHUNK
            return pltpu.async_copy(
                table2d.at[idx_v.at[p // B - r0, pl.ds(p % B, CHUNK)]],
                rows_v.at[i % NBUF],
                gsem,
            )

        copies = [gather_chunk(i) for i in range(min(NBUF, NCHUNK))]
        for i in range(NCHUNK):
            copies[i % NBUF].wait()
            out_cp = pltpu.async_copy(
                rows_v.at[i % NBUF],
                out_hbm.at[pl.ds(base + i * CHUNK, CHUNK)],
                osem,
            )
            out_cp.wait()
            if i + NBUF < NCHUNK:
                copies[i % NBUF] = gather_chunk(i + NBUF)

    return _gather


def _transpose_idx(idx, interpret=False):
    def body(src, dst):
        dst[:T, :] = src[...].T

    return pl.pallas_call(
        body,
        out_shape=jax.ShapeDtypeStruct((TPAD, B), jnp.int32),
        interpret=interpret,
    )(idx)


def _gru_step(x, h, wih, whh, bih, bhh):
    gi = jnp.dot(x, wih, preferred_element_type=jnp.float32) + bih
    gh = jnp.dot(h, whh, preferred_element_type=jnp.float32) + bhh
    r = jax.nn.sigmoid(gi[:, :HID] + gh[:, :HID])
    z = jax.nn.sigmoid(gi[:, HID:2 * HID] + gh[:, HID:2 * HID])
    n = jnp.tanh(gi[:, 2 * HID:] + r * gh[:, 2 * HID:])
    return (1.0 - z) * n + z * h


def _gru_body(x0_ref, x1_ref, hin_ref, _prev_ref, wih_ref, whh_ref,
              bih_ref, bhh_ref, out_ref, hout_ref, h_ref):
    t2 = pl.program_id(0)

    @pl.when(t2 == 0)
    def _():
        h_ref[...] = hin_ref[...]

    wih = wih_ref[...]
    whh = whh_ref[...]
    bih = bih_ref[...]
    bhh = bhh_ref[...]
    h0 = _gru_step(x0_ref[...], h_ref[...], wih, whh, bih, bhh)
    h1 = _gru_step(x1_ref[...], h0, wih, whh, bih, bhh)
    out_ref[:, :HID] = h0
    out_ref[:, HID:] = h1
    h_ref[...] = h1
    hout_ref[...] = h1


@functools.cache
def _make_gru(q, interpret=False):
    return pl.pallas_call(
        _gru_body,
        grid=(TQ // 2,),
        in_specs=[
            pl.BlockSpec((B, EMB), lambda t2: (2 * t2, 0)),
            pl.BlockSpec((B, EMB), lambda t2: (2 * t2 + 1, 0)),
            pl.BlockSpec((B, HID), lambda t2: (0, 0)),
            pl.BlockSpec(memory_space=pl.ANY),
            pl.BlockSpec((EMB, 3 * HID), lambda t2: (0, 0)),
            pl.BlockSpec((HID, 3 * HID), lambda t2: (0, 0)),
            pl.BlockSpec((1, 3 * HID), lambda t2: (0, 0)),
            pl.BlockSpec((1, 3 * HID), lambda t2: (0, 0)),
        ],
        out_specs=[
            pl.BlockSpec((B, 2 * HID), lambda t2, _q=q: (0, _q * (TQ // 2) + t2)),
            pl.BlockSpec((B, HID), lambda t2: (0, 0)),
        ],
        out_shape=[
            jax.ShapeDtypeStruct((B, T * HID), jnp.float32),
            jax.ShapeDtypeStruct((B, HID), jnp.float32),
        ],
        scratch_shapes=[pltpu.VMEM((B, HID), jnp.float32)],
        input_output_aliases={3: 0},
        interpret=interpret,
    )


@jax.jit
def kernel(input, table, W_ih, W_hh, b_ih, b_hh):
    idx_tm = _transpose_idx(input)
    wih_t = W_ih.T
    whh_t = W_hh.T
    bih = b_ih[None]
    bhh = b_hh[None]
    h = jnp.zeros((B, HID), jnp.float32)
    out2d = jnp.zeros((B, T * HID), jnp.float32)
    for q in range(Q):
        emb_q = _make_gather(q)(idx_tm, table)
        out2d, h = _make_gru(q)(emb_q, emb_q, h, out2d,
                                wih_t, whh_t, bih, bhh)
    out = out2d.reshape(B, T, HID)
    h_n = h[None]
    return out, h_n


# speedup vs baseline: 1.0047x; 1.0047x over previous
import functools

import jax
import jax.numpy as jnp
from jax import lax
from jax.experimental import pallas as pl
from jax.experimental.pallas import tpu as pltpu
from jax.experimental.pallas import tpu_sc as plsc

NUM_EMB = 1000000
EMB = 32
HID = 64
B = 4096
T = 50
TPAD = 56

NC = 2
NS = 16
NW = NC * NS

Q = 5
TQ = T // Q
NTOK = B * TQ
BPW = NTOK // NW
CHUNK = 256
NCHUNK = BPW // CHUNK
NBUF = 5


@functools.cache
def _make_gather(q):
    @functools.partial(
        pl.kernel,
        out_type=jax.ShapeDtypeStruct((NTOK, EMB), jnp.float32),
        mesh=plsc.VectorSubcoreMesh(core_axis_name="c", subcore_axis_name="s"),
        compiler_params=pltpu.CompilerParams(use_tc_tiling_on_sc=False),
        scratch_types=[
            pltpu.VMEM((BPW,), jnp.int32),
            pltpu.VMEM((NBUF, CHUNK, EMB), jnp.float32),
            pltpu.SemaphoreType.DMA,
            pltpu.SemaphoreType.DMA,
        ],
    )
    def _gather(idx_hbm, table2d, out_hbm, idx_v, rows_v, gsem, osem):
        wid = lax.axis_index("s") * NC + lax.axis_index("c")
        base = wid * BPW
        pltpu.sync_copy(idx_hbm.at[pl.ds(q * NTOK + base, BPW)], idx_v)

        def gather_chunk(i):
            return pltpu.async_copy(
                table2d.at[idx_v.at[pl.ds(i * CHUNK, CHUNK)]],
                rows_v.at[i % NBUF],
                gsem,
            )

        copies = [gather_chunk(i) for i in range(min(NBUF, NCHUNK))]
        for i in range(NCHUNK):
            copies[i % NBUF].wait()
            out_cp = pltpu.async_copy(
                rows_v.at[i % NBUF],
                out_hbm.at[pl.ds(base + i * CHUNK, CHUNK)],
                osem,
            )
            out_cp.wait()
            if i + NBUF < NCHUNK:
                copies[i % NBUF] = gather_chunk(i + NBUF)

    return _gather


def _transpose_idx(idx, interpret=False):
    def body(src, dst):
        dst[...] = src[...].T.reshape(B * T)

    return pl.pallas_call(
        body,
        out_shape=jax.ShapeDtypeStruct((B * T,), jnp.int32),
        interpret=interpret,
    )(idx)


def _gru_step(x, h, wih, whh, bih, bhh):
    gi = jnp.dot(x, wih, preferred_element_type=jnp.float32) + bih
    gh = jnp.dot(h, whh, preferred_element_type=jnp.float32) + bhh
    r = jax.nn.sigmoid(gi[:, :HID] + gh[:, :HID])
    z = jax.nn.sigmoid(gi[:, HID:2 * HID] + gh[:, HID:2 * HID])
    n = jnp.tanh(gi[:, 2 * HID:] + r * gh[:, 2 * HID:])
    return (1.0 - z) * n + z * h


def _gru_body(x0_ref, x1_ref, hin_ref, _prev_ref, wih_ref, whh_ref,
              bih_ref, bhh_ref, out_ref, hout_ref, h_ref):
    t2 = pl.program_id(0)

    @pl.when(t2 == 0)
    def _():
        h_ref[...] = hin_ref[...]

    wih = wih_ref[...]
    whh = whh_ref[...]
    bih = bih_ref[...]
    bhh = bhh_ref[...]
    h0 = _gru_step(x0_ref[...], h_ref[...], wih, whh, bih, bhh)
    h1 = _gru_step(x1_ref[...], h0, wih, whh, bih, bhh)
    out_ref[:, :HID] = h0
    out_ref[:, HID:] = h1
    h_ref[...] = h1
    hout_ref[...] = h1


@functools.cache
def _make_gru(q, interpret=False):
    return pl.pallas_call(
        _gru_body,
        grid=(TQ // 2,),
        in_specs=[
            pl.BlockSpec((B, EMB), lambda t2: (2 * t2, 0)),
            pl.BlockSpec((B, EMB), lambda t2: (2 * t2 + 1, 0)),
            pl.BlockSpec((B, HID), lambda t2: (0, 0)),
            pl.BlockSpec(memory_space=pl.ANY),
            pl.BlockSpec((EMB, 3 * HID), lambda t2: (0, 0)),
            pl.BlockSpec((HID, 3 * HID), lambda t2: (0, 0)),
            pl.BlockSpec((1, 3 * HID), lambda t2: (0, 0)),
            pl.BlockSpec((1, 3 * HID), lambda t2: (0, 0)),
        ],
        out_specs=[
            pl.BlockSpec((B, 2 * HID), lambda t2, _q=q: (0, _q * (TQ // 2) + t2)),
            pl.BlockSpec((B, HID), lambda t2: (0, 0)),
        ],
        out_shape=[
            jax.ShapeDtypeStruct((B, T * HID), jnp.float32),
            jax.ShapeDtypeStruct((B, HID), jnp.float32),
        ],
        scratch_shapes=[pltpu.VMEM((B, HID), jnp.float32)],
        input_output_aliases={3: 0},
        interpret=interpret,
    )


@jax.jit
def kernel(input, table, W_ih, W_hh, b_ih, b_hh):
    idx_tm = _transpose_idx(input)
    wih_t = W_ih.T
    whh_t = W_hh.T
    bih = b_ih[None]
    bhh = b_hh[None]
    h = jnp.zeros((B, HID), jnp.float32)
    out2d = jnp.zeros((B, T * HID), jnp.float32)
    for q in range(Q):
        emb_q = _make_gather(q)(idx_tm, table)
        out2d, h = _make_gru(q)(emb_q, emb_q, h, out2d,
                                wih_t, whh_t, bih, bhh)
    out = out2d.reshape(B, T, HID)
    h_n = h[None]
    return out, h_n
